# baseline (speedup 1.0000x reference)
import functools

import jax
import jax.numpy as jnp
from jax import lax
from jax.experimental import pallas as pl
from jax.experimental.pallas import tpu as pltpu
from jax.experimental.pallas import tpu_sc as plsc

_NUM_SAMPLES = 16
_B, _T, _C, _H, _W = 4, 64, 3, 224, 224
_PLANES = _B * _NUM_SAMPLES * _C
_HSPLIT = 2
_CHUNKS = _PLANES * _HSPLIT
_HH = _H // _HSPLIT
_NBUF = 4
_STAGGER = 2
_N0 = 16
_N1 = (_CHUNKS - 16 * _N0) // 16


def _coords(g):
    plane = g // _HSPLIT
    half = g % _HSPLIT
    b = plane // (_NUM_SAMPLES * _C)
    r = plane % (_NUM_SAMPLES * _C)
    s = r // _C
    c = r % _C
    t = (s * (_T - 1)) // (_NUM_SAMPLES - 1)
    return b, s, c, t, half * _HH


def _sc_body(x_hbm, o_hbm, bufs, in_sems, out_sems):
    cid = lax.axis_index("c")
    sid = lax.axis_index("s")

    def run(base, n):
        in_c = [None] * n
        out_c = [None] * n

        def start_out(k):
            b, s, c, _, h0 = _coords(base + k)
            kb = k % _NBUF
            in_c[k].wait()
            out_c[k] = pltpu.async_copy(
                bufs.at[kb], o_hbm.at[b, s, c, pl.ds(h0, _HH)],
                out_sems.at[kb])

        for k in range(n):
            kb = k % _NBUF
            if k >= _NBUF:
                out_c[k - _NBUF].wait()
            b, s, c, t, h0 = _coords(base + k)
            in_c[k] = pltpu.async_copy(
                x_hbm.at[b, t, c, pl.ds(h0, _HH)], bufs.at[kb],
                in_sems.at[kb])
            if k >= _STAGGER:
                start_out(k - _STAGGER)
        for k in range(n - _STAGGER, n):
            start_out(k)
        for k in range(n - _NBUF, n):
            out_c[k].wait()

    @pl.when(cid == 0)
    def _():
        run(sid * _N0, _N0)

    @pl.when(cid == 1)
    def _():
        run(16 * _N0 + sid * _N1, _N1)


@jax.jit
def kernel(x):
    mesh = plsc.VectorSubcoreMesh(core_axis_name="c", subcore_axis_name="s")
    f = functools.partial(
        pl.kernel,
        out_type=jax.ShapeDtypeStruct((_B, _NUM_SAMPLES, _C, _H, _W), x.dtype),
        mesh=mesh,
        scratch_types=[
            pltpu.VMEM((_NBUF, _HH, _W), jnp.float32),
            pltpu.SemaphoreType.DMA((_NBUF,)),
            pltpu.SemaphoreType.DMA((_NBUF,)),
        ],
    )(_sc_body)
    return f(x)


# speedup vs baseline: 1.0112x; 1.0112x over previous
import functools

import jax
import jax.numpy as jnp
from jax import lax
from jax.experimental import pallas as pl
from jax.experimental.pallas import tpu as pltpu
from jax.experimental.pallas import tpu_sc as plsc

_NUM_SAMPLES = 16
_B, _T, _C, _H, _W = 4, 64, 3, 224, 224
_PLANES = _B * _NUM_SAMPLES * _C
_HSPLIT = 2
_CHUNKS = _PLANES * _HSPLIT
_HH = _H // _HSPLIT
_NBUF = 4
_STAGGER = 2
_N0 = 8
_N1 = (_CHUNKS - 16 * _N0) // 16


def _coords(g):
    plane = g // _HSPLIT
    half = g % _HSPLIT
    b = plane // (_NUM_SAMPLES * _C)
    r = plane % (_NUM_SAMPLES * _C)
    s = r // _C
    c = r % _C
    t = (s * (_T - 1)) // (_NUM_SAMPLES - 1)
    return b, s, c, t, half * _HH


def _sc_body(x_hbm, o_hbm, bufs, in_sems, out_sems):
    cid = lax.axis_index("c")
    sid = lax.axis_index("s")

    def run(base, n):
        in_c = [None] * n
        out_c = [None] * n

        def start_out(k):
            b, s, c, _, h0 = _coords(base + k)
            kb = k % _NBUF
            in_c[k].wait()
            out_c[k] = pltpu.async_copy(
                bufs.at[kb], o_hbm.at[b, s, c, pl.ds(h0, _HH)],
                out_sems.at[kb])

        for k in range(n):
            kb = k % _NBUF
            if k >= _NBUF:
                out_c[k - _NBUF].wait()
            b, s, c, t, h0 = _coords(base + k)
            in_c[k] = pltpu.async_copy(
                x_hbm.at[b, t, c, pl.ds(h0, _HH)], bufs.at[kb],
                in_sems.at[kb])
            if k >= _STAGGER:
                start_out(k - _STAGGER)
        for k in range(n - _STAGGER, n):
            start_out(k)
        for k in range(n - _NBUF, n):
            out_c[k].wait()

    @pl.when(cid == 0)
    def _():
        run(sid * _N0, _N0)

    @pl.when(cid == 1)
    def _():
        run(16 * _N0 + sid * _N1, _N1)


@jax.jit
def kernel(x):
    mesh = plsc.VectorSubcoreMesh(core_axis_name="c", subcore_axis_name="s")
    f = functools.partial(
        pl.kernel,
        out_type=jax.ShapeDtypeStruct((_B, _NUM_SAMPLES, _C, _H, _W), x.dtype),
        mesh=mesh,
        scratch_types=[
            pltpu.VMEM((_NBUF, _HH, _W), jnp.float32),
            pltpu.SemaphoreType.DMA((_NBUF,)),
            pltpu.SemaphoreType.DMA((_NBUF,)),
        ],
    )(_sc_body)
    return f(x)


# speedup vs baseline: 1.0727x; 1.0608x over previous
import functools

import jax
import jax.numpy as jnp
from jax import lax
from jax.experimental import pallas as pl
from jax.experimental.pallas import tpu as pltpu
from jax.experimental.pallas import tpu_sc as plsc

_NUM_SAMPLES = 16
_B, _T, _C, _H, _W = 4, 64, 3, 224, 224
_PLANES = _B * _NUM_SAMPLES * _C
_HSPLIT = 2
_CHUNKS = _PLANES * _HSPLIT
_HH = _H // _HSPLIT
_NBUF = 4
_STAGGER = 2
_N0 = 12
_N1 = (_CHUNKS - 16 * _N0) // 16


def _coords(g):
    plane = g // _HSPLIT
    half = g % _HSPLIT
    b = plane // (_NUM_SAMPLES * _C)
    r = plane % (_NUM_SAMPLES * _C)
    s = r // _C
    c = r % _C
    t = (s * (_T - 1)) // (_NUM_SAMPLES - 1)
    return b, s, c, t, half * _HH


def _sc_body(x_hbm, o_hbm, bufs, in_sems, out_sems):
    cid = lax.axis_index("c")
    sid = lax.axis_index("s")

    def run(base, n):
        in_c = [None] * n
        out_c = [None] * n

        def start_out(k):
            b, s, c, _, h0 = _coords(base + k)
            kb = k % _NBUF
            in_c[k].wait()
            out_c[k] = pltpu.async_copy(
                bufs.at[kb], o_hbm.at[b, s, c, pl.ds(h0, _HH)],
                out_sems.at[kb])

        for k in range(n):
            kb = k % _NBUF
            if k >= _NBUF:
                out_c[k - _NBUF].wait()
            b, s, c, t, h0 = _coords(base + k)
            in_c[k] = pltpu.async_copy(
                x_hbm.at[b, t, c, pl.ds(h0, _HH)], bufs.at[kb],
                in_sems.at[kb])
            if k >= _STAGGER:
                start_out(k - _STAGGER)
        for k in range(n - _STAGGER, n):
            start_out(k)
        for k in range(n - _NBUF, n):
            out_c[k].wait()

    @pl.when(cid == 0)
    def _():
        run(sid * _N0, _N0)

    @pl.when(cid == 1)
    def _():
        run(16 * _N0 + sid * _N1, _N1)


@jax.jit
def kernel(x):
    mesh = plsc.VectorSubcoreMesh(core_axis_name="c", subcore_axis_name="s")
    f = functools.partial(
        pl.kernel,
        out_type=jax.ShapeDtypeStruct((_B, _NUM_SAMPLES, _C, _H, _W), x.dtype),
        mesh=mesh,
        scratch_types=[
            pltpu.VMEM((_NBUF, _HH, _W), jnp.float32),
            pltpu.SemaphoreType.DMA((_NBUF,)),
            pltpu.SemaphoreType.DMA((_NBUF,)),
        ],
    )(_sc_body)
    return f(x)
